# baseline (speedup 1.0000x reference)
import functools

import jax
import jax.numpy as jnp
from jax import lax
from jax.experimental import pallas as pl
from jax.experimental.pallas import tpu as pltpu
from jax.experimental.pallas import tpu_sc as plsc

_NUM_CORES = 2
_NUM_SUBCORES = 16
_NUM_WORKERS = _NUM_CORES * _NUM_SUBCORES
_CHUNK = 128


@functools.lru_cache(maxsize=None)
def _make_pipeline(B, D):
    n_total_chunks = B // _CHUNK
    b_per_w = B // _NUM_WORKERS
    n_chunks = b_per_w // _CHUNK
    mesh = plsc.VectorSubcoreMesh(core_axis_name="c", subcore_axis_name="s")

    @functools.partial(
        pl.kernel,
        mesh=mesh,
        out_type=jax.ShapeDtypeStruct((n_total_chunks, _CHUNK, D), jnp.float32),
        scratch_types=[
            pltpu.VMEM((n_chunks, _CHUNK), jnp.int32),
            pltpu.VMEM((n_chunks, _CHUNK, D), jnp.float32),
            pltpu.SemaphoreType.DMA,
        ],
        compiler_params=pltpu.CompilerParams(use_tc_tiling_on_sc=False),
    )
    def sc_gather(table_hbm, idx_hbm, out_hbm, idx_v, rows_v, sem):
        wid = lax.axis_index("s") * _NUM_CORES + lax.axis_index("c")
        pltpu.sync_copy(idx_hbm.at[pl.ds(wid * n_chunks, n_chunks)], idx_v)
        copies = [
            pltpu.async_copy(table_hbm.at[idx_v.at[c]], rows_v.at[c], sem)
            for c in range(n_chunks)
        ]
        for c in range(n_chunks):
            copies[c].wait()
            pltpu.sync_copy(rows_v.at[c], out_hbm.at[wid * n_chunks + c])

    def tc_body(y_ref, z_ref):
        z_ref[:, :D] = y_ref[:, :D].T
        z_ref[:, D:] = y_ref[:, D:].T

    tc_transpose = pl.pallas_call(
        tc_body,
        grid=(n_total_chunks,),
        in_specs=[pl.BlockSpec((_CHUNK // 2, 2 * D), lambda i: (i, 0))],
        out_specs=pl.BlockSpec((D, _CHUNK), lambda i: (0, i)),
        out_shape=jax.ShapeDtypeStruct((D, B), jnp.float32),
    )

    def run(x, embedding_weights):
        idx2d = (
            x.reshape(n_total_chunks, 2, _CHUNK // 2)
            .swapaxes(1, 2)
            .reshape(n_total_chunks, _CHUNK)
        )
        y3 = sc_gather(embedding_weights, idx2d)
        z = tc_transpose(y3.reshape(B // 2, 2 * D))
        return z.T

    return run


def kernel(x, embedding_weights):
    B = x.shape[1]
    D = embedding_weights.shape[1]
    return _make_pipeline(B, D)(x, embedding_weights)


# speedup vs baseline: 2.6669x; 2.6669x over previous
import functools

import jax
import jax.numpy as jnp
from jax import lax
from jax.experimental import pallas as pl
from jax.experimental.pallas import tpu as pltpu
from jax.experimental.pallas import tpu_sc as plsc

_NUM_CORES = 2
_NUM_SUBCORES = 16
_NUM_WORKERS = _NUM_CORES * _NUM_SUBCORES
_CHUNK = 128


@functools.lru_cache(maxsize=None)
def _make_gather(V, B, D):
    b_per_w = B // _NUM_WORKERS
    n_chunks = b_per_w // _CHUNK
    mesh = plsc.VectorSubcoreMesh(core_axis_name="c", subcore_axis_name="s")

    @functools.partial(
        pl.kernel,
        mesh=mesh,
        out_type=jax.ShapeDtypeStruct((B, D), jnp.float32),
        scratch_types=[
            pltpu.VMEM((n_chunks, _CHUNK), jnp.int32),
            pltpu.VMEM((b_per_w, D), jnp.float32),
            pltpu.VMEM_SHARED((V, D), jnp.float32),
            pltpu.SemaphoreType.DMA,
        ],
        compiler_params=pltpu.CompilerParams(use_tc_tiling_on_sc=False),
    )
    def gather(table_hbm, idx_hbm, out_hbm, idx_v, rows_v, table_sp, sem):
        sid = lax.axis_index("s")
        wid = sid * _NUM_CORES + lax.axis_index("c")
        base = wid * b_per_w
        @pl.when(sid == 0)
        def _():
            pltpu.sync_copy(table_hbm, table_sp)
        pltpu.sync_copy(idx_hbm.at[pl.ds(wid * n_chunks, n_chunks)], idx_v)
        plsc.subcore_barrier()
        copies = [
            pltpu.async_copy(
                table_sp.at[idx_v.at[c]],
                rows_v.at[pl.ds(c * _CHUNK, _CHUNK)],
                sem,
            )
            for c in range(n_chunks)
        ]
        for cp in copies:
            cp.wait()
        pltpu.sync_copy(rows_v, out_hbm.at[pl.ds(base, b_per_w)])

    return gather


def kernel(x, embedding_weights):
    V, D = embedding_weights.shape
    B = x.shape[1]
    idx2d = x.reshape(B // _CHUNK, _CHUNK)
    return _make_gather(V, B, D)(embedding_weights, idx2d)


# speedup vs baseline: 2.6977x; 1.0115x over previous
import functools

import jax
import jax.numpy as jnp
from jax import lax
from jax.experimental import pallas as pl
from jax.experimental.pallas import tpu as pltpu
from jax.experimental.pallas import tpu_sc as plsc

_NUM_CORES = 2
_NUM_SUBCORES = 16
_NUM_WORKERS = _NUM_CORES * _NUM_SUBCORES
_CHUNK = 128


@functools.lru_cache(maxsize=None)
def _make_gather(V, B, D):
    b_per_w = B // _NUM_WORKERS
    n_chunks = b_per_w // _CHUNK
    mesh = plsc.VectorSubcoreMesh(core_axis_name="c", subcore_axis_name="s")

    @functools.partial(
        pl.kernel,
        mesh=mesh,
        out_type=jax.ShapeDtypeStruct((B, D), jnp.float32),
        scratch_types=[
            pltpu.VMEM((n_chunks, _CHUNK), jnp.int32),
            pltpu.VMEM((b_per_w, D), jnp.float32),
            pltpu.VMEM_SHARED((V, D), jnp.float32),
            pltpu.SemaphoreType.DMA((4,)),
            pltpu.SemaphoreType.DMA,
        ],
        compiler_params=pltpu.CompilerParams(use_tc_tiling_on_sc=False),
    )
    def gather(table_hbm, idx_hbm, out_hbm, idx_v, rows_v, table_sp, gsem, wsem):
        sid = lax.axis_index("s")
        wid = sid * _NUM_CORES + lax.axis_index("c")
        base = wid * b_per_w
        @pl.when(sid == 0)
        def _():
            pltpu.sync_copy(table_hbm, table_sp)
        pltpu.sync_copy(idx_hbm.at[pl.ds(wid * n_chunks, n_chunks)], idx_v)
        plsc.subcore_barrier()
        copies = [
            pltpu.async_copy(
                table_sp.at[idx_v.at[c]],
                rows_v.at[pl.ds(c * _CHUNK, _CHUNK)],
                gsem.at[c],
            )
            for c in range(n_chunks)
        ]
        writes = []
        for c in range(n_chunks):
            copies[c].wait()
            writes.append(
                pltpu.async_copy(
                    rows_v.at[pl.ds(c * _CHUNK, _CHUNK)],
                    out_hbm.at[pl.ds(base + c * _CHUNK, _CHUNK)],
                    wsem,
                )
            )
        for w in writes:
            w.wait()

    return gather


def kernel(x, embedding_weights):
    V, D = embedding_weights.shape
    B = x.shape[1]
    idx2d = x.reshape(B // _CHUNK, _CHUNK)
    return _make_gather(V, B, D)(embedding_weights, idx2d)
